# baseline (speedup 1.0000x reference)
import functools

import jax
import jax.numpy as jnp
from jax import lax
from jax.experimental import pallas as pl
from jax.experimental.pallas import tpu as pltpu
from jax.experimental.pallas import tpu_sc as plsc

VOCAB = 32
EMBED = 16
N_TOKENS = 4 * 8192

_info = plsc.get_sparse_core_info()
_NC, _NS, _L = _info.num_cores, _info.num_subcores, _info.num_lanes
_NW = _NC * _NS
_TPW = N_TOKENS // _NW

_sc_mesh = plsc.VectorSubcoreMesh(core_axis_name="c", subcore_axis_name="s")


@functools.partial(
    pl.kernel,
    mesh=_sc_mesh,
    compiler_params=pltpu.CompilerParams(needs_layout_passes=False),
    out_type=jax.ShapeDtypeStruct((_NW, VOCAB, VOCAB), jnp.float32),
    scratch_types=[
        pltpu.VMEM((_TPW,), jnp.int32),
        pltpu.VMEM((_TPW,), jnp.int32),
        pltpu.VMEM((VOCAB, VOCAB), jnp.float32),
        pltpu.SemaphoreType.DMA,
        pltpu.SemaphoreType.DMA,
    ],
)
def _sc_pair_hist(idx_hbm, tgt_hbm, out_hbm, idx_v, tgt_v, hist_v, sem1, sem2):
    wid = lax.axis_index("s") * _NC + lax.axis_index("c")
    base = wid * _TPW
    cp_i = pltpu.async_copy(idx_hbm.at[pl.ds(base, _TPW)], idx_v, sem1)
    cp_t = pltpu.async_copy(tgt_hbm.at[pl.ds(base, _TPW)], tgt_v, sem2)

    zeros = jnp.zeros((_L,), jnp.float32)
    for r in range(VOCAB):
        for c in range(0, VOCAB, _L):
            hist_v[r, pl.ds(c, _L)] = zeros

    cp_i.wait()
    cp_t.wait()

    ones = jnp.ones((_L,), jnp.float32)

    def body(i, carry):
        s = i * _L
        iv = idx_v[pl.ds(s, _L)]
        tv = tgt_v[pl.ds(s, _L)]
        plsc.addupdate_scatter(hist_v, [iv, tv], ones)
        return carry

    lax.fori_loop(0, _TPW // _L, body, 0)
    pltpu.sync_copy(hist_v, out_hbm.at[wid])


def _combine_body(hist_ref, embed_ref, w_ref, b_ref, out_ref):
    hsum = jnp.sum(hist_ref[...], axis=0)
    table = (
        jnp.dot(embed_ref[...], w_ref[...], preferred_element_type=jnp.float32)
        + b_ref[...]
    )
    m = jnp.max(table, axis=1, keepdims=True)
    lse = m + jnp.log(jnp.sum(jnp.exp(table - m), axis=1, keepdims=True))
    nll = lse - table
    loss = jnp.sum(hsum * nll, keepdims=True) * (1.0 / N_TOKENS)
    out_ref[...] = loss.reshape(1, 1)


def _combine(hist, embed, W, b):
    return pl.pallas_call(
        _combine_body,
        out_shape=jax.ShapeDtypeStruct((1, 1), jnp.float32),
    )(hist, embed, W, b.reshape(1, VOCAB))


def kernel(idx, targets, embed, W, b):
    hist = _sc_pair_hist(idx.reshape(-1), targets.reshape(-1))
    return _combine(hist, embed, W, b).reshape(())


# speedup vs baseline: 1.0029x; 1.0029x over previous
import functools

import jax
import jax.numpy as jnp
from jax import lax
from jax.experimental import pallas as pl
from jax.experimental.pallas import tpu as pltpu
from jax.experimental.pallas import tpu_sc as plsc

VOCAB = 32
EMBED = 16
N_TOKENS = 4 * 8192

_info = plsc.get_sparse_core_info()
_NC, _NS, _L = _info.num_cores, _info.num_subcores, _info.num_lanes
_NW = _NC * _NS
_TPW = N_TOKENS // _NW

_sc_mesh = plsc.VectorSubcoreMesh(core_axis_name="c", subcore_axis_name="s")


@functools.partial(
    pl.kernel,
    mesh=_sc_mesh,
    compiler_params=pltpu.CompilerParams(
        needs_layout_passes=False, skip_device_barrier=True
    ),
    out_type=jax.ShapeDtypeStruct((_NW, VOCAB, VOCAB), jnp.float32),
    scratch_types=[
        pltpu.VMEM((_TPW,), jnp.int32),
        pltpu.VMEM((_TPW,), jnp.int32),
        pltpu.VMEM((VOCAB, VOCAB), jnp.float32),
        pltpu.SemaphoreType.DMA,
        pltpu.SemaphoreType.DMA,
    ],
)
def _sc_pair_hist(idx_hbm, tgt_hbm, out_hbm, idx_v, tgt_v, hist_v, sem1, sem2):
    wid = lax.axis_index("s") * _NC + lax.axis_index("c")
    base = wid * _TPW
    cp_i = pltpu.async_copy(idx_hbm.at[pl.ds(base, _TPW)], idx_v, sem1)
    cp_t = pltpu.async_copy(tgt_hbm.at[pl.ds(base, _TPW)], tgt_v, sem2)

    zeros = jnp.zeros((_L,), jnp.float32)
    for r in range(VOCAB):
        for c in range(0, VOCAB, _L):
            hist_v[r, pl.ds(c, _L)] = zeros

    cp_i.wait()
    cp_t.wait()

    ones = jnp.ones((_L,), jnp.float32)

    def body(i, carry):
        s = i * _L
        iv = idx_v[pl.ds(s, _L)]
        tv = tgt_v[pl.ds(s, _L)]
        plsc.addupdate_scatter(hist_v, [iv, tv], ones)
        return carry

    lax.fori_loop(0, _TPW // _L, body, 0)
    pltpu.sync_copy(hist_v, out_hbm.at[wid])


def _combine_body(hist_ref, embed_ref, w_ref, b_ref, out_ref):
    hsum = jnp.sum(hist_ref[...], axis=0)
    table = (
        jnp.dot(embed_ref[...], w_ref[...], preferred_element_type=jnp.float32)
        + b_ref[...]
    )
    m = jnp.max(table, axis=1, keepdims=True)
    lse = m + jnp.log(jnp.sum(jnp.exp(table - m), axis=1, keepdims=True))
    nll = lse - table
    loss = jnp.sum(hsum * nll, keepdims=True) * (1.0 / N_TOKENS)
    out_ref[...] = loss.reshape(1, 1)


def _combine(hist, embed, W, b):
    return pl.pallas_call(
        _combine_body,
        out_shape=jax.ShapeDtypeStruct((1, 1), jnp.float32),
    )(hist, embed, W, b.reshape(1, VOCAB))


def kernel(idx, targets, embed, W, b):
    hist = _sc_pair_hist(idx.reshape(-1), targets.reshape(-1))
    return _combine(hist, embed, W, b).reshape(())
